# baseline (speedup 1.0000x reference)
import functools

import jax
import jax.numpy as jnp
from jax import lax
from jax.experimental import pallas as pl
from jax.experimental.pallas import tpu as pltpu
from jax.experimental.pallas import tpu_sc as plsc

N = 10000
E = 320000
D = 128
L_LAYERS = 3

NP = 10240
TRASH = N
B = 80
NW = 32
EPW = E // NW
NCH = EPW // B
NBUF = 4
ROWS_PER_TILE = NP // 16


def _seg_body(cur_hbm, src_hbm, dst_hbm, out_hbm,
              sidx, didx, rows, zbuf, acc, gsems, isems, ssems):
    c = lax.axis_index("c")
    s = lax.axis_index("s")
    wid = c * 16 + s
    base_e = wid * EPW

    def fire_idx(t, slot):
        off = base_e + t * B
        pltpu.async_copy(src_hbm.at[pl.ds(off, B)], sidx.at[slot], isems.at[slot])
        pltpu.async_copy(dst_hbm.at[pl.ds(off, B)], didx.at[slot], isems.at[slot])

    def wait_idx(t, slot):
        off = base_e + t * B
        pltpu.make_async_copy(src_hbm.at[pl.ds(off, B)], sidx.at[slot],
                              isems.at[slot]).wait()
        pltpu.make_async_copy(dst_hbm.at[pl.ds(off, B)], didx.at[slot],
                              isems.at[slot]).wait()

    def fix_dst(slot):
        for j in range(B // 16):
            sl = pl.ds(j * 16, 16)
            sv = sidx[slot, sl]
            dv = didx[slot, sl]
            didx[slot, sl] = jnp.where(sv == dv, TRASH, dv)

    def fire_gather(slot):
        pltpu.async_copy(cur_hbm.at[sidx.at[slot]], rows.at[slot], gsems.at[slot])

    def wait_gather(slot):
        pltpu.make_async_copy(cur_hbm.at[sidx.at[slot]], rows.at[slot],
                              gsems.at[slot]).wait()

    def fire_scatter(slot):
        pltpu.async_copy(rows.at[slot], acc.at[didx.at[slot]], ssems.at[slot],
                         add=True)

    def wait_scatter(slot):
        pltpu.make_async_copy(rows.at[slot], acc.at[didx.at[slot]],
                              ssems.at[slot]).wait()

    zero16 = jnp.zeros((16,), jnp.float32)
    for i in range(16):
        for j in range(D // 16):
            zbuf[i, pl.ds(j * 16, 16)] = zero16
    row0 = s * ROWS_PER_TILE

    fire_idx(0, 0)
    fire_idx(1, 1)

    def zero_chunk(k, carry):
        pltpu.sync_copy(zbuf, acc.at[pl.ds(row0 + k * 16, 16)])
        return carry

    lax.fori_loop(0, ROWS_PER_TILE // 16, zero_chunk, 0)

    wait_idx(0, 0)
    fix_dst(0)
    fire_gather(0)
    plsc.subcore_barrier()

    def group(g, carry):
        for b in range(NBUF):
            t = g * NBUF + b
            s1 = (b + 1) % NBUF
            s2 = (b + 2) % NBUF

            @pl.when(t + 1 < NCH)
            def _():
                wait_idx(t + 1, s1)
                fix_dst(s1)
                fire_gather(s1)

            @pl.when(t + 2 < NCH)
            def _():
                @pl.when(t >= 2)
                def _():
                    wait_scatter(s2)
                fire_idx(t + 2, s2)

            @pl.when(t < NCH)
            def _():
                wait_gather(b)
                fire_scatter(b)
        return carry

    lax.fori_loop(0, (NCH + NBUF - 1) // NBUF, group, 0)
    for tt in range(NCH - NBUF, NCH):
        wait_scatter(tt % NBUF)
    plsc.subcore_barrier()

    @pl.when(row0 < N)
    def _writeback():
        nwb = ROWS_PER_TILE // B
        for k in range(nwb):
            slot = k % 2
            r0 = row0 + k * B
            if k >= 2:
                rp = row0 + (k - 2) * B
                pltpu.make_async_copy(rows.at[slot],
                                      out_hbm.at[c, pl.ds(rp, B)],
                                      gsems.at[slot]).wait()
            pltpu.sync_copy(acc.at[pl.ds(r0, B)], rows.at[slot])
            pltpu.async_copy(rows.at[slot], out_hbm.at[c, pl.ds(r0, B)],
                             gsems.at[slot])
        for k in range(nwb - 2, nwb):
            slot = k % 2
            r0 = row0 + k * B
            pltpu.make_async_copy(rows.at[slot],
                                  out_hbm.at[c, pl.ds(r0, B)],
                                  gsems.at[slot]).wait()


def _segment_sum_sc(cur, src, dst):
    mesh = plsc.VectorSubcoreMesh(core_axis_name="c", subcore_axis_name="s")
    f = functools.partial(
        pl.kernel,
        mesh=mesh,
        out_type=jax.ShapeDtypeStruct((2, NP, D), jnp.float32),
        scratch_types=[
            pltpu.VMEM((NBUF, B), jnp.int32),
            pltpu.VMEM((NBUF, B), jnp.int32),
            pltpu.VMEM((NBUF, B, D), jnp.float32),
            pltpu.VMEM((16, D), jnp.float32),
            pltpu.VMEM_SHARED((NP, D), jnp.float32),
            pltpu.SemaphoreType.DMA((NBUF,)),
            pltpu.SemaphoreType.DMA((NBUF,)),
            pltpu.SemaphoreType.DMA((NBUF,)),
        ],
    )(_seg_body)
    return f(cur, src, dst)


def _pre_body(x_ref, w_ref, b_ref, jk_ref, h_ref):
    h = (
        jnp.dot(x_ref[...], w_ref[...], precision=lax.Precision.HIGHEST,
                preferred_element_type=jnp.float32)
        + b_ref[...]
    )
    jk_ref[...] = h
    h_ref[...] = h


def _layer_body(jk_ref, p0_ref, p1_ref, wc_ref, wua_ref, wub_ref, bc_ref,
                bu_ref, jko_ref, cur_ref):
    aggr = p0_ref[0] + p1_ref[0]
    conv = jnp.dot(aggr, wc_ref[...], precision=lax.Precision.HIGHEST,
                   preferred_element_type=jnp.float32) + bc_ref[...]
    upd = (
        jnp.dot(conv, wua_ref[...], precision=lax.Precision.HIGHEST,
                preferred_element_type=jnp.float32)
        + jnp.dot(jk_ref[...], wub_ref[...], precision=lax.Precision.HIGHEST,
                  preferred_element_type=jnp.float32)
        + bu_ref[...]
    )
    act = jnp.maximum(upd, 0.0)
    jko_ref[...] = act
    cur_ref[...] = act


_BR = 400
_JKD = (L_LAYERS + 1) * D


def _tc_pre(x, W_pre, b_pre):
    grid = (N // _BR,)
    return pl.pallas_call(
        _pre_body,
        grid=grid,
        in_specs=[
            pl.BlockSpec((_BR, D), lambda i: (i, 0)),
            pl.BlockSpec((D, D), lambda i: (0, 0)),
            pl.BlockSpec((1, D), lambda i: (0, 0)),
        ],
        out_specs=[
            pl.BlockSpec((_BR, D), lambda i: (i, 0)),
            pl.BlockSpec((_BR, D), lambda i: (i, 0)),
        ],
        out_shape=[
            jax.ShapeDtypeStruct((N, _JKD), jnp.float32),
            jax.ShapeDtypeStruct((N, D), jnp.float32),
        ],
    )(x, W_pre, b_pre.reshape(1, D))


def _tc_layer(jk, partial, Wc_l, bc_l, Wu_l, bu_l, l):
    grid = (N // _BR,)
    return pl.pallas_call(
        _layer_body,
        grid=grid,
        in_specs=[
            pl.BlockSpec((_BR, D), lambda i: (i, l)),
            pl.BlockSpec((1, _BR, D), lambda i: (0, i, 0)),
            pl.BlockSpec((1, _BR, D), lambda i: (1, i, 0)),
            pl.BlockSpec((D, D), lambda i: (0, 0)),
            pl.BlockSpec((D, D), lambda i: (0, 0)),
            pl.BlockSpec((D, D), lambda i: (0, 0)),
            pl.BlockSpec((1, D), lambda i: (0, 0)),
            pl.BlockSpec((1, D), lambda i: (0, 0)),
        ],
        out_specs=[
            pl.BlockSpec((_BR, D), lambda i: (i, l + 1)),
            pl.BlockSpec((_BR, D), lambda i: (i, 0)),
        ],
        out_shape=[
            jax.ShapeDtypeStruct((N, _JKD), jnp.float32),
            jax.ShapeDtypeStruct((N, D), jnp.float32),
        ],
        input_output_aliases={0: 0},
    )(jk, partial, partial, Wc_l, Wu_l[:D], Wu_l[D:], bc_l.reshape(1, D),
      bu_l.reshape(1, D))


def kernel(x, edge_index, W_pre, b_pre, Wc, bc, Wu, bu):
    src = edge_index[0]
    dst = edge_index[1]
    jk, cur = _tc_pre(x, W_pre, b_pre)
    for l in range(L_LAYERS):
        partial = _segment_sum_sc(cur, src, dst)
        jk, cur = _tc_layer(jk, partial, Wc[l], bc[l], Wu[l], bu[l], l)
    return jk


# speedup vs baseline: 1.0238x; 1.0238x over previous
import functools

import jax
import jax.numpy as jnp
from jax import lax
from jax.experimental import pallas as pl
from jax.experimental.pallas import tpu as pltpu
from jax.experimental.pallas import tpu_sc as plsc

N = 10000
E = 320000
D = 128
L_LAYERS = 3

NP = 10240
TRASH = N
B = 80
NW = 32
EPW = E // NW
NCH = EPW // B
NBUF = 4
ROWS_PER_TILE = NP // 16


def _seg_body(cur_hbm, src_hbm, dst_hbm, out_hbm,
              sidx, didx, rows, zbuf, acc, gsems, isems, ssems):
    c = lax.axis_index("c")
    s = lax.axis_index("s")
    wid = c * 16 + s
    base_e = wid * EPW

    def fire_idx(t, slot):
        off = base_e + t * B
        pltpu.async_copy(src_hbm.at[pl.ds(off, B)], sidx.at[slot], isems.at[slot])
        pltpu.async_copy(dst_hbm.at[pl.ds(off, B)], didx.at[slot], isems.at[slot])

    def wait_idx(t, slot):
        off = base_e + t * B
        pltpu.make_async_copy(src_hbm.at[pl.ds(off, B)], sidx.at[slot],
                              isems.at[slot]).wait()
        pltpu.make_async_copy(dst_hbm.at[pl.ds(off, B)], didx.at[slot],
                              isems.at[slot]).wait()

    def fire_gather(slot):
        pltpu.async_copy(cur_hbm.at[sidx.at[slot]], rows.at[slot], gsems.at[slot])

    def wait_gather(slot):
        pltpu.make_async_copy(cur_hbm.at[sidx.at[slot]], rows.at[slot],
                              gsems.at[slot]).wait()

    def fire_scatter(slot):
        pltpu.async_copy(rows.at[slot], acc.at[didx.at[slot]], ssems.at[slot],
                         add=True)

    def wait_scatter(slot):
        pltpu.make_async_copy(rows.at[slot], acc.at[didx.at[slot]],
                              ssems.at[slot]).wait()

    zero16 = jnp.zeros((16,), jnp.float32)
    for i in range(16):
        for j in range(D // 16):
            zbuf[i, pl.ds(j * 16, 16)] = zero16
    row0 = s * ROWS_PER_TILE

    fire_idx(0, 0)
    fire_idx(1, 1)

    def zero_chunk(k, carry):
        pltpu.sync_copy(zbuf, acc.at[pl.ds(row0 + k * 16, 16)])
        return carry

    lax.fori_loop(0, ROWS_PER_TILE // 16, zero_chunk, 0)

    wait_idx(0, 0)
    fire_gather(0)
    plsc.subcore_barrier()

    def group(g, carry):
        for b in range(NBUF):
            t = g * NBUF + b
            s1 = (b + 1) % NBUF
            s2 = (b + 2) % NBUF

            @pl.when(t + 1 < NCH)
            def _():
                wait_idx(t + 1, s1)
                fire_gather(s1)

            @pl.when(t + 2 < NCH)
            def _():
                @pl.when(t >= 2)
                def _():
                    wait_scatter(s2)
                fire_idx(t + 2, s2)

            @pl.when(t < NCH)
            def _():
                wait_gather(b)
                fire_scatter(b)
        return carry

    lax.fori_loop(0, (NCH + NBUF - 1) // NBUF, group, 0)
    for tt in range(NCH - NBUF, NCH):
        wait_scatter(tt % NBUF)
    plsc.subcore_barrier()

    @pl.when(row0 < N)
    def _writeback():
        nwb = ROWS_PER_TILE // B
        for k in range(nwb):
            slot = k % 2
            r0 = row0 + k * B
            if k >= 2:
                rp = row0 + (k - 2) * B
                pltpu.make_async_copy(rows.at[slot],
                                      out_hbm.at[c, pl.ds(rp, B)],
                                      gsems.at[slot]).wait()
            pltpu.sync_copy(acc.at[pl.ds(r0, B)], rows.at[slot])
            pltpu.async_copy(rows.at[slot], out_hbm.at[c, pl.ds(r0, B)],
                             gsems.at[slot])
        for k in range(nwb - 2, nwb):
            slot = k % 2
            r0 = row0 + k * B
            pltpu.make_async_copy(rows.at[slot],
                                  out_hbm.at[c, pl.ds(r0, B)],
                                  gsems.at[slot]).wait()


def _segment_sum_sc(cur, src, dst):
    mesh = plsc.VectorSubcoreMesh(core_axis_name="c", subcore_axis_name="s")
    f = functools.partial(
        pl.kernel,
        mesh=mesh,
        out_type=jax.ShapeDtypeStruct((2, NP, D), jnp.float32),
        scratch_types=[
            pltpu.VMEM((NBUF, B), jnp.int32),
            pltpu.VMEM((NBUF, B), jnp.int32),
            pltpu.VMEM((NBUF, B, D), jnp.float32),
            pltpu.VMEM((16, D), jnp.float32),
            pltpu.VMEM_SHARED((NP, D), jnp.float32),
            pltpu.SemaphoreType.DMA((NBUF,)),
            pltpu.SemaphoreType.DMA((NBUF,)),
            pltpu.SemaphoreType.DMA((NBUF,)),
        ],
    )(_seg_body)
    return f(cur, src, dst)


def _pre_body(x_ref, w_ref, b_ref, e_ref, jk_ref, h_ref, d2_ref):
    h = (
        jnp.dot(x_ref[...], w_ref[...], precision=lax.Precision.DEFAULT,
                preferred_element_type=jnp.float32)
        + b_ref[...]
    )
    jk_ref[...] = h
    h_ref[...] = h
    sv = e_ref[0:1, :]
    dv = e_ref[1:2, :]
    d2_ref[...] = jnp.where(sv == dv, TRASH, dv)


def _layer_body(jk_ref, p0_ref, p1_ref, wc_ref, wua_ref, wub_ref, bc_ref,
                bu_ref, jko_ref, cur_ref):
    aggr = p0_ref[0] + p1_ref[0]
    conv = jnp.dot(aggr, wc_ref[...], precision=lax.Precision.DEFAULT,
                   preferred_element_type=jnp.float32) + bc_ref[...]
    upd = (
        jnp.dot(conv, wua_ref[...], precision=lax.Precision.DEFAULT,
                preferred_element_type=jnp.float32)
        + jnp.dot(jk_ref[...], wub_ref[...], precision=lax.Precision.DEFAULT,
                  preferred_element_type=jnp.float32)
        + bu_ref[...]
    )
    act = jnp.maximum(upd, 0.0)
    jko_ref[...] = act
    cur_ref[...] = act


_BR = 400
_JKD = (L_LAYERS + 1) * D


_EB = E // (N // _BR)


def _tc_pre(x, W_pre, b_pre, edge_index):
    grid = (N // _BR,)
    return pl.pallas_call(
        _pre_body,
        grid=grid,
        in_specs=[
            pl.BlockSpec((_BR, D), lambda i: (i, 0)),
            pl.BlockSpec((D, D), lambda i: (0, 0)),
            pl.BlockSpec((1, D), lambda i: (0, 0)),
            pl.BlockSpec((2, _EB), lambda i: (0, i)),
        ],
        out_specs=[
            pl.BlockSpec((_BR, D), lambda i: (i, 0)),
            pl.BlockSpec((_BR, D), lambda i: (i, 0)),
            pl.BlockSpec((1, _EB), lambda i: (0, i)),
        ],
        out_shape=[
            jax.ShapeDtypeStruct((N, _JKD), jnp.float32),
            jax.ShapeDtypeStruct((N, D), jnp.float32),
            jax.ShapeDtypeStruct((1, E), jnp.int32),
        ],
    )(x, W_pre, b_pre.reshape(1, D), edge_index)


def _tc_layer(jk, partial, Wc_l, bc_l, Wu_l, bu_l, l):
    grid = (N // _BR,)
    return pl.pallas_call(
        _layer_body,
        grid=grid,
        in_specs=[
            pl.BlockSpec((_BR, D), lambda i: (i, l)),
            pl.BlockSpec((1, _BR, D), lambda i: (0, i, 0)),
            pl.BlockSpec((1, _BR, D), lambda i: (1, i, 0)),
            pl.BlockSpec((D, D), lambda i: (0, 0)),
            pl.BlockSpec((D, D), lambda i: (0, 0)),
            pl.BlockSpec((D, D), lambda i: (0, 0)),
            pl.BlockSpec((1, D), lambda i: (0, 0)),
            pl.BlockSpec((1, D), lambda i: (0, 0)),
        ],
        out_specs=[
            pl.BlockSpec((_BR, D), lambda i: (i, l + 1)),
            pl.BlockSpec((_BR, D), lambda i: (i, 0)),
        ],
        out_shape=[
            jax.ShapeDtypeStruct((N, _JKD), jnp.float32),
            jax.ShapeDtypeStruct((N, D), jnp.float32),
        ],
        input_output_aliases={0: 0},
    )(jk, partial, partial, Wc_l, Wu_l[:D], Wu_l[D:], bc_l.reshape(1, D),
      bu_l.reshape(1, D))


def kernel(x, edge_index, W_pre, b_pre, Wc, bc, Wu, bu):
    src = edge_index[0]
    jk, cur, dst2 = _tc_pre(x, W_pre, b_pre, edge_index)
    dst = dst2.reshape(E)
    for l in range(L_LAYERS):
        partial = _segment_sum_sc(cur, src, dst)
        jk, cur = _tc_layer(jk, partial, Wc[l], bc[l], Wu[l], bu[l], l)
    return jk


# speedup vs baseline: 1.1021x; 1.0765x over previous
import functools

import jax
import jax.numpy as jnp
from jax import lax
from jax.experimental import pallas as pl
from jax.experimental.pallas import tpu as pltpu
from jax.experimental.pallas import tpu_sc as plsc

N = 10000
E = 320000
D = 128
L_LAYERS = 3

NP = 10240
TRASH = N
B = 80
NW = 32
EPW = E // NW
NCH = EPW // B
NBUF = 4
ROWS_PER_TILE = NP // 16
ZR = 16


def _seg_body(cur_hbm, src_hbm, dst_hbm, out_hbm,
              sidx, didx, rows, zbuf, acc, gsems, isems, ssems):
    c = lax.axis_index("c")
    s = lax.axis_index("s")
    wid = c * 16 + s
    base_e = wid * EPW

    def fire_idx(t, slot):
        off = base_e + t * B
        pltpu.async_copy(src_hbm.at[pl.ds(off, B)], sidx.at[slot], isems.at[slot])
        pltpu.async_copy(dst_hbm.at[pl.ds(off, B)], didx.at[slot], isems.at[slot])

    def wait_idx(t, slot):
        off = base_e + t * B
        pltpu.make_async_copy(src_hbm.at[pl.ds(off, B)], sidx.at[slot],
                              isems.at[slot]).wait()
        pltpu.make_async_copy(dst_hbm.at[pl.ds(off, B)], didx.at[slot],
                              isems.at[slot]).wait()

    def fire_gather(slot):
        pltpu.async_copy(cur_hbm.at[sidx.at[slot]], rows.at[slot], gsems.at[slot])

    def wait_gather(slot):
        pltpu.make_async_copy(cur_hbm.at[sidx.at[slot]], rows.at[slot],
                              gsems.at[slot]).wait()

    def fire_scatter(slot):
        pltpu.async_copy(rows.at[slot], acc.at[didx.at[slot]], ssems.at[slot],
                         add=True)

    def wait_scatter(slot):
        pltpu.make_async_copy(rows.at[slot], acc.at[didx.at[slot]],
                              ssems.at[slot]).wait()

    zero16 = jnp.zeros((16,), jnp.float32)
    for i in range(ZR):
        for j in range(D // 16):
            zbuf[i, pl.ds(j * 16, 16)] = zero16
    row0 = s * ROWS_PER_TILE

    fire_idx(0, 0)
    fire_idx(1, 1)

    for k in range(ROWS_PER_TILE // ZR):
        pltpu.sync_copy(zbuf, acc.at[pl.ds(row0 + k * ZR, ZR)])

    wait_idx(0, 0)
    fire_gather(0)
    plsc.subcore_barrier()

    def group(g, carry):
        for b in range(NBUF):
            t = g * NBUF + b
            s1 = (b + 1) % NBUF
            s2 = (b + 2) % NBUF

            @pl.when(t + 1 < NCH)
            def _():
                wait_idx(t + 1, s1)
                fire_gather(s1)

            @pl.when(t + 2 < NCH)
            def _():
                @pl.when(t >= 2)
                def _():
                    wait_scatter(s2)
                fire_idx(t + 2, s2)

            @pl.when(t < NCH)
            def _():
                wait_gather(b)
                fire_scatter(b)
        return carry

    lax.fori_loop(0, (NCH + NBUF - 1) // NBUF, group, 0)
    for tt in range(NCH - NBUF, NCH):
        wait_scatter(tt % NBUF)
    plsc.subcore_barrier()

    @pl.when(row0 < N)
    def _writeback():
        nwb = ROWS_PER_TILE // B
        for k in range(nwb):
            slot = k % 2
            r0 = row0 + k * B
            if k >= 2:
                rp = row0 + (k - 2) * B
                pltpu.make_async_copy(rows.at[slot],
                                      out_hbm.at[c, pl.ds(rp, B)],
                                      gsems.at[slot]).wait()
            pltpu.sync_copy(acc.at[pl.ds(r0, B)], rows.at[slot])
            pltpu.async_copy(rows.at[slot], out_hbm.at[c, pl.ds(r0, B)],
                             gsems.at[slot])
        for k in range(nwb - 2, nwb):
            slot = k % 2
            r0 = row0 + k * B
            pltpu.make_async_copy(rows.at[slot],
                                  out_hbm.at[c, pl.ds(r0, B)],
                                  gsems.at[slot]).wait()


def _segment_sum_sc(cur, src, dst):
    mesh = plsc.VectorSubcoreMesh(core_axis_name="c", subcore_axis_name="s")
    f = functools.partial(
        pl.kernel,
        mesh=mesh,
        out_type=jax.ShapeDtypeStruct((2, NP, D), jnp.float32),
        scratch_types=[
            pltpu.VMEM((NBUF, B), jnp.int32),
            pltpu.VMEM((NBUF, B), jnp.int32),
            pltpu.VMEM((NBUF, B, D), jnp.float32),
            pltpu.VMEM((ZR, D), jnp.float32),
            pltpu.VMEM_SHARED((NP, D), jnp.float32),
            pltpu.SemaphoreType.DMA((NBUF,)),
            pltpu.SemaphoreType.DMA((NBUF,)),
            pltpu.SemaphoreType.DMA((NBUF,)),
        ],
    )(_seg_body)
    return f(cur, src, dst)


def _pre_body(x_ref, w_ref, b_ref, e_ref, jk_ref, h_ref, d2_ref):
    h = (
        jnp.dot(x_ref[...], w_ref[...], precision=lax.Precision.DEFAULT,
                preferred_element_type=jnp.float32)
        + b_ref[...]
    )
    jk_ref[...] = h
    h_ref[...] = h
    sv = e_ref[0:1, :]
    dv = e_ref[1:2, :]
    d2_ref[...] = jnp.where(sv == dv, TRASH, dv)


def _layer_body(jk_ref, p0_ref, p1_ref, wc_ref, wua_ref, wub_ref, bc_ref,
                bu_ref, jko_ref, cur_ref):
    aggr = p0_ref[0] + p1_ref[0]
    conv = jnp.dot(aggr, wc_ref[...], precision=lax.Precision.DEFAULT,
                   preferred_element_type=jnp.float32) + bc_ref[...]
    upd = (
        jnp.dot(conv, wua_ref[...], precision=lax.Precision.DEFAULT,
                preferred_element_type=jnp.float32)
        + jnp.dot(jk_ref[...], wub_ref[...], precision=lax.Precision.DEFAULT,
                  preferred_element_type=jnp.float32)
        + bu_ref[...]
    )
    act = jnp.maximum(upd, 0.0)
    jko_ref[...] = act
    cur_ref[...] = act


_BR = 1000
_JKD = (L_LAYERS + 1) * D


_EB = E // (N // _BR)


def _tc_pre(x, W_pre, b_pre, edge_index):
    grid = (N // _BR,)
    return pl.pallas_call(
        _pre_body,
        grid=grid,
        in_specs=[
            pl.BlockSpec((_BR, D), lambda i: (i, 0)),
            pl.BlockSpec((D, D), lambda i: (0, 0)),
            pl.BlockSpec((1, D), lambda i: (0, 0)),
            pl.BlockSpec((2, _EB), lambda i: (0, i)),
        ],
        out_specs=[
            pl.BlockSpec((_BR, D), lambda i: (i, 0)),
            pl.BlockSpec((_BR, D), lambda i: (i, 0)),
            pl.BlockSpec((1, _EB), lambda i: (0, i)),
        ],
        out_shape=[
            jax.ShapeDtypeStruct((N, _JKD), jnp.float32),
            jax.ShapeDtypeStruct((N, D), jnp.float32),
            jax.ShapeDtypeStruct((1, E), jnp.int32),
        ],
    )(x, W_pre, b_pre.reshape(1, D), edge_index)


def _tc_layer(jk, partial, Wc_l, bc_l, Wu_l, bu_l, l):
    grid = (N // _BR,)
    return pl.pallas_call(
        _layer_body,
        grid=grid,
        in_specs=[
            pl.BlockSpec((_BR, D), lambda i: (i, l)),
            pl.BlockSpec((1, _BR, D), lambda i: (0, i, 0)),
            pl.BlockSpec((1, _BR, D), lambda i: (1, i, 0)),
            pl.BlockSpec((D, D), lambda i: (0, 0)),
            pl.BlockSpec((D, D), lambda i: (0, 0)),
            pl.BlockSpec((D, D), lambda i: (0, 0)),
            pl.BlockSpec((1, D), lambda i: (0, 0)),
            pl.BlockSpec((1, D), lambda i: (0, 0)),
        ],
        out_specs=[
            pl.BlockSpec((_BR, D), lambda i: (i, l + 1)),
            pl.BlockSpec((_BR, D), lambda i: (i, 0)),
        ],
        out_shape=[
            jax.ShapeDtypeStruct((N, _JKD), jnp.float32),
            jax.ShapeDtypeStruct((N, D), jnp.float32),
        ],
        input_output_aliases={0: 0},
    )(jk, partial, partial, Wc_l, Wu_l[:D], Wu_l[D:], bc_l.reshape(1, D),
      bu_l.reshape(1, D))


def kernel(x, edge_index, W_pre, b_pre, Wc, bc, Wu, bu):
    src = edge_index[0]
    jk, cur, dst2 = _tc_pre(x, W_pre, b_pre, edge_index)
    dst = dst2.reshape(E)
    for l in range(L_LAYERS):
        partial = _segment_sum_sc(cur, src, dst)
        jk, cur = _tc_layer(jk, partial, Wc[l], bc[l], Wu[l], bu[l], l)
    return jk


# speedup vs baseline: 1.1393x; 1.0337x over previous
import functools

import jax
import jax.numpy as jnp
from jax import lax
from jax.experimental import pallas as pl
from jax.experimental.pallas import tpu as pltpu
from jax.experimental.pallas import tpu_sc as plsc

N = 10000
E = 320000
D = 128
L_LAYERS = 3

NP = 10240
TRASH = N
B = 80
NW = 32
EPW = E // NW
NCH = EPW // B
NBUF = 4
ROWS_PER_TILE = NP // 16
ZR = 16


def _seg_body(cur_hbm, src_hbm, dst_hbm, out_hbm,
              sidx, didx, rows, zbuf, acc, gsems, isems, ssems):
    c = lax.axis_index("c")
    s = lax.axis_index("s")
    wid = c * 16 + s
    base_e = wid * EPW

    def fire_idx(t, slot):
        off = base_e + t * B
        pltpu.async_copy(src_hbm.at[pl.ds(off, B)], sidx.at[slot], isems.at[slot])
        pltpu.async_copy(dst_hbm.at[pl.ds(off, B)], didx.at[slot], isems.at[slot])

    def wait_idx(t, slot):
        off = base_e + t * B
        pltpu.make_async_copy(src_hbm.at[pl.ds(off, B)], sidx.at[slot],
                              isems.at[slot]).wait()
        pltpu.make_async_copy(dst_hbm.at[pl.ds(off, B)], didx.at[slot],
                              isems.at[slot]).wait()

    def fire_gather(slot):
        pltpu.async_copy(cur_hbm.at[sidx.at[slot]], rows.at[slot], gsems.at[slot])

    def wait_gather(slot):
        pltpu.make_async_copy(cur_hbm.at[sidx.at[slot]], rows.at[slot],
                              gsems.at[slot]).wait()

    def fire_scatter(slot):
        pltpu.async_copy(rows.at[slot], acc.at[didx.at[slot]], ssems.at[slot],
                         add=True)

    def wait_scatter(slot):
        pltpu.make_async_copy(rows.at[slot], acc.at[didx.at[slot]],
                              ssems.at[slot]).wait()

    zero16 = jnp.zeros((16,), jnp.float32)
    for i in range(ZR):
        for j in range(D // 16):
            zbuf[i, pl.ds(j * 16, 16)] = zero16
    row0 = s * ROWS_PER_TILE

    fire_idx(0, 0)
    fire_idx(1, 1)
    wait_idx(0, 0)
    fire_gather(0)

    for k in range(ROWS_PER_TILE // ZR):
        pltpu.sync_copy(zbuf, acc.at[pl.ds(row0 + k * ZR, ZR)])

    plsc.subcore_barrier()

    def group(g, carry):
        for b in range(NBUF):
            t = g * NBUF + b
            s1 = (b + 1) % NBUF
            s2 = (b + 2) % NBUF

            @pl.when(t + 1 < NCH)
            def _():
                wait_idx(t + 1, s1)
                fire_gather(s1)

            @pl.when(t + 2 < NCH)
            def _():
                @pl.when(t >= 2)
                def _():
                    wait_scatter(s2)
                fire_idx(t + 2, s2)

            @pl.when(t < NCH)
            def _():
                wait_gather(b)
                fire_scatter(b)
        return carry

    lax.fori_loop(0, (NCH + NBUF - 1) // NBUF, group, 0)
    for tt in range(NCH - NBUF, NCH):
        wait_scatter(tt % NBUF)
    plsc.subcore_barrier()

    @pl.when(row0 < N)
    def _writeback():
        nwb = ROWS_PER_TILE // B
        for k in range(nwb):
            slot = k % 2
            r0 = row0 + k * B
            if k >= 2:
                rp = row0 + (k - 2) * B
                pltpu.make_async_copy(rows.at[slot],
                                      out_hbm.at[c, pl.ds(rp, B)],
                                      gsems.at[slot]).wait()
            pltpu.sync_copy(acc.at[pl.ds(r0, B)], rows.at[slot])
            pltpu.async_copy(rows.at[slot], out_hbm.at[c, pl.ds(r0, B)],
                             gsems.at[slot])
        for k in range(nwb - 2, nwb):
            slot = k % 2
            r0 = row0 + k * B
            pltpu.make_async_copy(rows.at[slot],
                                  out_hbm.at[c, pl.ds(r0, B)],
                                  gsems.at[slot]).wait()


def _segment_sum_sc(cur, src, dst):
    mesh = plsc.VectorSubcoreMesh(core_axis_name="c", subcore_axis_name="s")
    f = functools.partial(
        pl.kernel,
        mesh=mesh,
        out_type=jax.ShapeDtypeStruct((2, NP, D), jnp.float32),
        scratch_types=[
            pltpu.VMEM((NBUF, B), jnp.int32),
            pltpu.VMEM((NBUF, B), jnp.int32),
            pltpu.VMEM((NBUF, B, D), jnp.float32),
            pltpu.VMEM((ZR, D), jnp.float32),
            pltpu.VMEM_SHARED((NP, D), jnp.float32),
            pltpu.SemaphoreType.DMA((NBUF,)),
            pltpu.SemaphoreType.DMA((NBUF,)),
            pltpu.SemaphoreType.DMA((NBUF,)),
        ],
    )(_seg_body)
    return f(cur, src, dst)


def _pre_body(x_ref, w_ref, b_ref, e_ref, jk_ref, h_ref, d2_ref):
    h = (
        jnp.dot(x_ref[...], w_ref[...], precision=lax.Precision.DEFAULT,
                preferred_element_type=jnp.float32)
        + b_ref[...]
    )
    jk_ref[...] = h
    h_ref[...] = h
    sv = e_ref[0:1, :]
    dv = e_ref[1:2, :]
    d2_ref[...] = jnp.where(sv == dv, TRASH, dv)


def _layer_body(jk_ref, p0_ref, p1_ref, wc_ref, wua_ref, wub_ref, bc_ref,
                bu_ref, jko_ref, cur_ref):
    aggr = p0_ref[0] + p1_ref[0]
    conv = jnp.dot(aggr, wc_ref[...], precision=lax.Precision.DEFAULT,
                   preferred_element_type=jnp.float32) + bc_ref[...]
    upd = (
        jnp.dot(conv, wua_ref[...], precision=lax.Precision.DEFAULT,
                preferred_element_type=jnp.float32)
        + jnp.dot(jk_ref[...], wub_ref[...], precision=lax.Precision.DEFAULT,
                  preferred_element_type=jnp.float32)
        + bu_ref[...]
    )
    act = jnp.maximum(upd, 0.0)
    jko_ref[...] = act
    cur_ref[...] = act


_BR = 2000
_JKD = (L_LAYERS + 1) * D


_EB = E // (N // _BR)


def _tc_pre(x, W_pre, b_pre, edge_index):
    grid = (N // _BR,)
    return pl.pallas_call(
        _pre_body,
        grid=grid,
        in_specs=[
            pl.BlockSpec((_BR, D), lambda i: (i, 0)),
            pl.BlockSpec((D, D), lambda i: (0, 0)),
            pl.BlockSpec((1, D), lambda i: (0, 0)),
            pl.BlockSpec((2, _EB), lambda i: (0, i)),
        ],
        out_specs=[
            pl.BlockSpec((_BR, D), lambda i: (i, 0)),
            pl.BlockSpec((_BR, D), lambda i: (i, 0)),
            pl.BlockSpec((1, _EB), lambda i: (0, i)),
        ],
        out_shape=[
            jax.ShapeDtypeStruct((N, _JKD), jnp.float32),
            jax.ShapeDtypeStruct((N, D), jnp.float32),
            jax.ShapeDtypeStruct((1, E), jnp.int32),
        ],
    )(x, W_pre, b_pre.reshape(1, D), edge_index)


def _tc_layer(jk, partial, Wc_l, bc_l, Wu_l, bu_l, l):
    grid = (N // _BR,)
    return pl.pallas_call(
        _layer_body,
        grid=grid,
        in_specs=[
            pl.BlockSpec((_BR, D), lambda i: (i, l)),
            pl.BlockSpec((1, _BR, D), lambda i: (0, i, 0)),
            pl.BlockSpec((1, _BR, D), lambda i: (1, i, 0)),
            pl.BlockSpec((D, D), lambda i: (0, 0)),
            pl.BlockSpec((D, D), lambda i: (0, 0)),
            pl.BlockSpec((D, D), lambda i: (0, 0)),
            pl.BlockSpec((1, D), lambda i: (0, 0)),
            pl.BlockSpec((1, D), lambda i: (0, 0)),
        ],
        out_specs=[
            pl.BlockSpec((_BR, D), lambda i: (i, l + 1)),
            pl.BlockSpec((_BR, D), lambda i: (i, 0)),
        ],
        out_shape=[
            jax.ShapeDtypeStruct((N, _JKD), jnp.float32),
            jax.ShapeDtypeStruct((N, D), jnp.float32),
        ],
        input_output_aliases={0: 0},
    )(jk, partial, partial, Wc_l, Wu_l[:D], Wu_l[D:], bc_l.reshape(1, D),
      bu_l.reshape(1, D))


def kernel(x, edge_index, W_pre, b_pre, Wc, bc, Wu, bu):
    src = edge_index[0]
    jk, cur, dst2 = _tc_pre(x, W_pre, b_pre, edge_index)
    dst = dst2.reshape(E)
    for l in range(L_LAYERS):
        partial = _segment_sum_sc(cur, src, dst)
        jk, cur = _tc_layer(jk, partial, Wc[l], bc[l], Wu[l], bu[l], l)
    return jk


# speedup vs baseline: 1.1574x; 1.0159x over previous
import functools

import jax
import jax.numpy as jnp
from jax import lax
from jax.experimental import pallas as pl
from jax.experimental.pallas import tpu as pltpu
from jax.experimental.pallas import tpu_sc as plsc

N = 10000
E = 320000
D = 128
L_LAYERS = 3

NP = 10240
TRASH = N
B = 80
NW = 32
EPW = E // NW
NCH = EPW // B
NBUF = 4
ROWS_PER_TILE = NP // 16
ZR = 16


def _seg_body(cur_hbm, src_hbm, dst_hbm, out_hbm,
              sidx, didx, rows, zbuf, acc, gsems, isems, ssems):
    c = lax.axis_index("c")
    s = lax.axis_index("s")
    wid = c * 16 + s
    base_e = wid * EPW

    def fire_idx(t, slot):
        off = base_e + t * B
        pltpu.async_copy(src_hbm.at[pl.ds(off, B)], sidx.at[slot], isems.at[slot])
        pltpu.async_copy(dst_hbm.at[pl.ds(off, B)], didx.at[slot], isems.at[slot])

    def wait_idx(t, slot):
        off = base_e + t * B
        pltpu.make_async_copy(src_hbm.at[pl.ds(off, B)], sidx.at[slot],
                              isems.at[slot]).wait()
        pltpu.make_async_copy(dst_hbm.at[pl.ds(off, B)], didx.at[slot],
                              isems.at[slot]).wait()

    def fire_gather(slot):
        pltpu.async_copy(cur_hbm.at[sidx.at[slot]], rows.at[slot], gsems.at[slot])

    def wait_gather(slot):
        pltpu.make_async_copy(cur_hbm.at[sidx.at[slot]], rows.at[slot],
                              gsems.at[slot]).wait()

    def fire_scatter(slot):
        pltpu.async_copy(rows.at[slot], acc.at[didx.at[slot]], ssems.at[slot],
                         add=True)

    def wait_scatter(slot):
        pltpu.make_async_copy(rows.at[slot], acc.at[didx.at[slot]],
                              ssems.at[slot]).wait()

    zero16 = jnp.zeros((16,), jnp.float32)
    for i in range(ZR):
        for j in range(D // 16):
            zbuf[i, pl.ds(j * 16, 16)] = zero16
    row0 = s * ROWS_PER_TILE

    fire_idx(0, 0)
    fire_idx(1, 1)
    wait_idx(0, 0)
    fire_gather(0)

    for k in range(ROWS_PER_TILE // ZR):
        pltpu.sync_copy(zbuf, acc.at[pl.ds(row0 + k * ZR, ZR)])

    plsc.subcore_barrier()

    def group(g, carry):
        for b in range(NBUF):
            t = g * NBUF + b
            s1 = (b + 1) % NBUF
            s2 = (b + 2) % NBUF

            @pl.when(t + 1 < NCH)
            def _():
                wait_idx(t + 1, s1)
                fire_gather(s1)

            @pl.when(t + 2 < NCH)
            def _():
                @pl.when(t >= 2)
                def _():
                    wait_scatter(s2)
                fire_idx(t + 2, s2)

            @pl.when(t < NCH)
            def _():
                wait_gather(b)
                fire_scatter(b)
        return carry

    lax.fori_loop(0, (NCH + NBUF - 1) // NBUF, group, 0)
    for tt in range(NCH - NBUF, NCH):
        wait_scatter(tt % NBUF)
    plsc.subcore_barrier()

    @pl.when(row0 < N)
    def _writeback():
        nwb = ROWS_PER_TILE // B
        for k in range(nwb):
            slot = k % 2
            r0 = row0 + k * B
            if k >= 2:
                rp = row0 + (k - 2) * B
                pltpu.make_async_copy(rows.at[slot],
                                      out_hbm.at[c, pl.ds(rp, B)],
                                      gsems.at[slot]).wait()
            pltpu.sync_copy(acc.at[pl.ds(r0, B)], rows.at[slot])
            pltpu.async_copy(rows.at[slot], out_hbm.at[c, pl.ds(r0, B)],
                             gsems.at[slot])
        for k in range(nwb - 2, nwb):
            slot = k % 2
            r0 = row0 + k * B
            pltpu.make_async_copy(rows.at[slot],
                                  out_hbm.at[c, pl.ds(r0, B)],
                                  gsems.at[slot]).wait()


def _segment_sum_sc(cur, src, dst):
    mesh = plsc.VectorSubcoreMesh(core_axis_name="c", subcore_axis_name="s")
    f = functools.partial(
        pl.kernel,
        mesh=mesh,
        out_type=jax.ShapeDtypeStruct((2, NP, D), jnp.float32),
        scratch_types=[
            pltpu.VMEM((NBUF, B), jnp.int32),
            pltpu.VMEM((NBUF, B), jnp.int32),
            pltpu.VMEM((NBUF, B, D), jnp.float32),
            pltpu.VMEM((ZR, D), jnp.float32),
            pltpu.VMEM_SHARED((NP, D), jnp.float32),
            pltpu.SemaphoreType.DMA((NBUF,)),
            pltpu.SemaphoreType.DMA((NBUF,)),
            pltpu.SemaphoreType.DMA((NBUF,)),
        ],
    )(_seg_body)
    return f(cur, src, dst)


def _pre_body(x_ref, w_ref, b_ref, e_ref, jk_ref, h_ref, d2_ref):
    h = (
        jnp.dot(x_ref[...], w_ref[...], precision=lax.Precision.DEFAULT,
                preferred_element_type=jnp.float32)
        + b_ref[...]
    )
    jk_ref[...] = h
    h_ref[...] = h
    sv = e_ref[0:1, :]
    dv = e_ref[1:2, :]
    d2_ref[...] = jnp.where(sv == dv, TRASH, dv)


def _layer_body(jk_ref, p0_ref, p1_ref, wc_ref, wua_ref, wub_ref, bc_ref,
                bu_ref, jko_ref, cur_ref):
    aggr = p0_ref[0] + p1_ref[0]
    conv = jnp.dot(aggr, wc_ref[...], precision=lax.Precision.DEFAULT,
                   preferred_element_type=jnp.float32) + bc_ref[...]
    upd = (
        jnp.dot(conv, wua_ref[...], precision=lax.Precision.DEFAULT,
                preferred_element_type=jnp.float32)
        + jnp.dot(jk_ref[...], wub_ref[...], precision=lax.Precision.DEFAULT,
                  preferred_element_type=jnp.float32)
        + bu_ref[...]
    )
    act = jnp.maximum(upd, 0.0)
    jko_ref[...] = act
    cur_ref[...] = act


_BR = 5000
_JKD = (L_LAYERS + 1) * D


_EB = E // (N // _BR)


def _tc_pre(x, W_pre, b_pre, edge_index):
    grid = (N // _BR,)
    return pl.pallas_call(
        _pre_body,
        grid=grid,
        in_specs=[
            pl.BlockSpec((_BR, D), lambda i: (i, 0)),
            pl.BlockSpec((D, D), lambda i: (0, 0)),
            pl.BlockSpec((1, D), lambda i: (0, 0)),
            pl.BlockSpec((2, _EB), lambda i: (0, i)),
        ],
        out_specs=[
            pl.BlockSpec((_BR, D), lambda i: (i, 0)),
            pl.BlockSpec((_BR, D), lambda i: (i, 0)),
            pl.BlockSpec((1, _EB), lambda i: (0, i)),
        ],
        out_shape=[
            jax.ShapeDtypeStruct((N, _JKD), jnp.float32),
            jax.ShapeDtypeStruct((N, D), jnp.float32),
            jax.ShapeDtypeStruct((1, E), jnp.int32),
        ],
    )(x, W_pre, b_pre.reshape(1, D), edge_index)


def _tc_layer(jk, partial, Wc_l, bc_l, Wu_l, bu_l, l):
    grid = (N // _BR,)
    return pl.pallas_call(
        _layer_body,
        grid=grid,
        in_specs=[
            pl.BlockSpec((_BR, D), lambda i: (i, l)),
            pl.BlockSpec((1, _BR, D), lambda i: (0, i, 0)),
            pl.BlockSpec((1, _BR, D), lambda i: (1, i, 0)),
            pl.BlockSpec((D, D), lambda i: (0, 0)),
            pl.BlockSpec((D, D), lambda i: (0, 0)),
            pl.BlockSpec((D, D), lambda i: (0, 0)),
            pl.BlockSpec((1, D), lambda i: (0, 0)),
            pl.BlockSpec((1, D), lambda i: (0, 0)),
        ],
        out_specs=[
            pl.BlockSpec((_BR, D), lambda i: (i, l + 1)),
            pl.BlockSpec((_BR, D), lambda i: (i, 0)),
        ],
        out_shape=[
            jax.ShapeDtypeStruct((N, _JKD), jnp.float32),
            jax.ShapeDtypeStruct((N, D), jnp.float32),
        ],
        input_output_aliases={0: 0},
    )(jk, partial, partial, Wc_l, Wu_l[:D], Wu_l[D:], bc_l.reshape(1, D),
      bu_l.reshape(1, D))


def kernel(x, edge_index, W_pre, b_pre, Wc, bc, Wu, bu):
    src = edge_index[0]
    jk, cur, dst2 = _tc_pre(x, W_pre, b_pre, edge_index)
    dst = dst2.reshape(E)
    for l in range(L_LAYERS):
        partial = _segment_sum_sc(cur, src, dst)
        jk, cur = _tc_layer(jk, partial, Wc[l], bc[l], Wu[l], bu[l], l)
    return jk
